# Initial kernel scaffold; baseline (speedup 1.0000x reference)
import jax
import jax.numpy as jnp
from jax.experimental import pallas as pl


def kernel(X, edge_idx, C):
    raise NotImplementedError("write your pallas kernel here")


# speedup vs baseline: 9.2165x; 9.2165x over previous
import jax
import jax.numpy as jnp
import numpy as np
from jax import lax
from jax.experimental import pallas as pl
from jax.experimental.pallas import tpu as pltpu
from jax.experimental.pallas import tpu_sc as plsc

N = 10000
K = 64
OUTW = 48
ROWW = 16
NC, NS = 2, 16
NW = NC * NS
NB = 8
EC = NB * K
NGRP = EC // 128
NCHUNK = N // NB
SCALE = 0.1

_F = np.arange(OUTW)
_PAT = np.zeros((8, 16), np.int32)
_PAT[0:3] = (_F % 12).reshape(3, 16)
_PAT[3:6] = (3 * (_F // 12) + _F % 3).reshape(3, 16)


def _body(x_hbm, e_hbm, c_hbm, pat_hbm, out_hbm, c_v, idx_v, rows_v, xi_v,
          s_v, out_v, pat_v, sem):
  wid = lax.axis_index("s") * NC + lax.axis_index("c")

  pltpu.sync_copy(c_hbm, c_v)
  pltpu.sync_copy(pat_hbm, pat_v)

  ia = [pat_v[r, :] for r in range(3)]
  ib = [pat_v[3 + r, :] for r in range(3)]
  zv = pat_v[6, :]

  def chunk_body(t, _):
    ch = wid + t * NW
    n0 = ch * NB
    pltpu.sync_copy(e_hbm.at[pl.ds(ch * NGRP, NGRP)], idx_v)
    copies = [
        pltpu.make_async_copy(
            x_hbm.at[idx_v.at[g]], rows_v.at[pl.ds(g * 128, 128)], sem)
        for g in range(NGRP)
    ]
    for cp in copies:
      cp.start()
    pltpu.sync_copy(x_hbm.at[pl.ds(n0, NB)], xi_v)
    for cp in copies:
      cp.wait()

    for i in range(NB):
      n = n0 + i
      bvecs = [plsc.load_gather(xi_v, [zv + i, ib[r]]) for r in range(3)]
      mi = plsc.load_gather(c_v, [zv + n])
      smi = (mi > 0).astype(jnp.float32) * SCALE
      for g in range(K // 16):
        e16 = idx_v[i // 2, pl.ds((i % 2) * K + g * 16, 16)]
        cj = plsc.load_gather(c_v, [e16])
        s_v[pl.ds(g * 16, 16)] = smi * (cj > 0).astype(jnp.float32)

      def _edge(e, _):
        row = i * K + e
        sv = plsc.load_gather(s_v, [zv + e])
        rsplat = zv + row
        for r in range(3):
          a = plsc.load_gather(rows_v, [rsplat, ia[r]])
          out_v[row, pl.ds(r * 16, 16)] = (a - bvecs[r]) * sv
        return 0

      lax.fori_loop(0, K, _edge, 0)

    pltpu.sync_copy(out_v, out_hbm.at[pl.ds(ch * EC, EC)])
    return 0

  nch = (NCHUNK - wid + NW - 1) // NW
  lax.fori_loop(0, nch, chunk_body, 0)


@jax.jit
def _run(x16, eidx2, c, pat):
  mesh = plsc.VectorSubcoreMesh(core_axis_name="c", subcore_axis_name="s")
  f = pl.kernel(
      _body,
      out_type=jax.ShapeDtypeStruct((N * K, OUTW), jnp.float32),
      mesh=mesh,
      compiler_params=pltpu.CompilerParams(
          use_tc_tiling_on_sc=False, needs_layout_passes=False),
      scratch_types=[
          pltpu.VMEM((N,), jnp.int32),
          pltpu.VMEM((NGRP, 128), jnp.int32),
          pltpu.VMEM((EC, ROWW), jnp.float32),
          pltpu.VMEM((NB, ROWW), jnp.float32),
          pltpu.VMEM((K,), jnp.float32),
          pltpu.VMEM((EC, OUTW), jnp.float32),
          pltpu.VMEM((8, 16), jnp.int32),
          pltpu.SemaphoreType.DMA,
      ],
  )
  return f(x16, eidx2, c, pat)


def kernel(X, edge_idx, C):
  B = X.shape[0]
  x16 = jnp.pad(X.reshape(N, 12), ((0, 0), (0, ROWW - 12)))
  eidx2 = edge_idx.reshape(N * K // 128, 128).astype(jnp.int32)
  c = C.reshape(N).astype(jnp.int32)
  out = _run(x16, eidx2, c, jnp.asarray(_PAT))
  return out.reshape(B, N, K, OUTW)


# speedup vs baseline: 13.3450x; 1.4480x over previous
import jax
import jax.numpy as jnp
import numpy as np
from jax import lax
from jax.experimental import pallas as pl
from jax.experimental.pallas import tpu as pltpu
from jax.experimental.pallas import tpu_sc as plsc

N = 10000
K = 64
OUTW = 48
ROWW = 16
NC, NS = 2, 16
NW = NC * NS
NB = 8
EC = NB * K
NGRP = EC // 128
NCHUNK = N // NB
SCALE = 0.1

_F = np.arange(OUTW)
_PAT = np.zeros((8, 16), np.int32)
_PAT[0:3] = (_F % 12).reshape(3, 16)
_PAT[3:6] = (3 * (_F // 12) + _F % 3).reshape(3, 16)


def _body(x_hbm, e_hbm, c_hbm, pat_hbm, out_hbm, c_v, idx_v, rows_v, xi_v,
          s_v, out_v, pat_v, sem):
  wid = lax.axis_index("s") * NC + lax.axis_index("c")

  pltpu.sync_copy(c_hbm, c_v)
  pltpu.sync_copy(pat_hbm, pat_v)

  ia = [pat_v[r, :] for r in range(3)]
  ib = [pat_v[3 + r, :] for r in range(3)]
  zv = pat_v[6, :]

  def chunk_body(t, _):
    ch = wid + t * NW
    n0 = ch * NB
    pltpu.sync_copy(e_hbm.at[pl.ds(ch * NGRP, NGRP)], idx_v)
    copies = [
        pltpu.make_async_copy(
            x_hbm.at[idx_v.at[g]], rows_v.at[pl.ds(g * 128, 128)], sem)
        for g in range(NGRP)
    ]
    for cp in copies:
      cp.start()
    pltpu.sync_copy(x_hbm.at[pl.ds(n0, NB)], xi_v)
    for cp in copies:
      cp.wait()

    for i in range(NB):
      n = n0 + i
      bvecs = [plsc.load_gather(xi_v, [zv + i, ib[r]]) for r in range(3)]
      mi = plsc.load_gather(c_v, [zv + n])
      smi = (mi > 0).astype(jnp.float32) * SCALE
      for g in range(K // 16):
        e16 = idx_v[i // 2, pl.ds((i % 2) * K + g * 16, 16)]
        cj = plsc.load_gather(c_v, [e16])
        s_v[pl.ds(g * 16, 16)] = smi * (cj > 0).astype(jnp.float32)

      @plsc.parallel_loop(0, K, 1, unroll=8)
      def _edge(e):
        row = i * K + e
        sv = plsc.load_gather(s_v, [zv + e])
        rsplat = zv + row
        for r in range(3):
          a = plsc.load_gather(rows_v, [rsplat, ia[r]])
          out_v[row, pl.ds(r * 16, 16)] = (a - bvecs[r]) * sv

    pltpu.sync_copy(out_v, out_hbm.at[pl.ds(ch * EC, EC)])
    return 0

  nch = (NCHUNK - wid + NW - 1) // NW
  lax.fori_loop(0, nch, chunk_body, 0)


@jax.jit
def _run(x16, eidx2, c, pat):
  mesh = plsc.VectorSubcoreMesh(core_axis_name="c", subcore_axis_name="s")
  f = pl.kernel(
      _body,
      out_type=jax.ShapeDtypeStruct((N * K, OUTW), jnp.float32),
      mesh=mesh,
      compiler_params=pltpu.CompilerParams(
          use_tc_tiling_on_sc=False, needs_layout_passes=False),
      scratch_types=[
          pltpu.VMEM((N,), jnp.int32),
          pltpu.VMEM((NGRP, 128), jnp.int32),
          pltpu.VMEM((EC, ROWW), jnp.float32),
          pltpu.VMEM((NB, ROWW), jnp.float32),
          pltpu.VMEM((K,), jnp.float32),
          pltpu.VMEM((EC, OUTW), jnp.float32),
          pltpu.VMEM((8, 16), jnp.int32),
          pltpu.SemaphoreType.DMA,
      ],
  )
  return f(x16, eidx2, c, pat)


def kernel(X, edge_idx, C):
  B = X.shape[0]
  x16 = jnp.pad(X.reshape(N, 12), ((0, 0), (0, ROWW - 12)))
  eidx2 = edge_idx.reshape(N * K // 128, 128).astype(jnp.int32)
  c = C.reshape(N).astype(jnp.int32)
  out = _run(x16, eidx2, c, jnp.asarray(_PAT))
  return out.reshape(B, N, K, OUTW)
